# Initial kernel scaffold; baseline (speedup 1.0000x reference)
import jax
import jax.numpy as jnp
from jax.experimental import pallas as pl


def kernel(D_ca, Angle_cab, id3_reduce_ca, Kidx):
    raise NotImplementedError("write your pallas kernel here")


# speedup vs baseline: 2.3022x; 2.3022x over previous
import functools

import numpy as np
import jax
import jax.numpy as jnp
from jax import lax
from jax.experimental import pallas as pl
from jax.experimental.pallas import tpu as pltpu
from jax.experimental.pallas import tpu_sc as plsc

_NUM_SPH = 7
_NUM_RAD = 6
_CUTOFF = 5.0
_ENV_EXP = 5
_N_EDGES = 640000
_N_TRIP = 1280000

_SC_NC = 2
_SC_NS = 16
_SC_NW = _SC_NC * _SC_NS
_B_PER_W = _N_TRIP // _SC_NW
_CHUNK = 80
_NCHUNK = _B_PER_W // _CHUNK


def _sjn(x, n):
    x = np.asarray(x, dtype=np.float64)
    j0 = np.sin(x) / x
    if n == 0:
        return j0
    j1 = np.sin(x) / x ** 2 - np.cos(x) / x
    jm1, jc = j0, j1
    for l in range(1, n):
        jm1, jc = jc, (2 * l + 1) / x * jc - jm1
    return jc


def _sjn_zeros(n, k):
    zeros = np.zeros((n, k), dtype=np.float64)
    zeros[0] = np.arange(1, k + 1) * np.pi
    points = np.arange(1, k + n) * np.pi
    for i in range(1, n):
        m = k + n - 1 - i
        racines = np.zeros(m, dtype=np.float64)
        for j in range(m):
            a, b = points[j], points[j + 1]
            fa = _sjn(a, i)
            for _ in range(100):
                mid = 0.5 * (a + b)
                fm = _sjn(mid, i)
                if fa * fm <= 0.0:
                    b = mid
                else:
                    a, fa = mid, fm
            racines[j] = 0.5 * (a + b)
        points = racines
        zeros[i, :k] = racines[:k]
    return zeros


_ZQ = _sjn_zeros(_NUM_SPH, _NUM_RAD)
_NRM = np.zeros((_NUM_SPH, _NUM_RAD), dtype=np.float64)
for _l in range(_NUM_SPH):
    for _n in range(_NUM_RAD):
        _NRM[_l, _n] = 1.0 / np.sqrt(0.5 * _sjn(_ZQ[_l, _n], _l + 1) ** 2)

_ZFLAT = _ZQ.reshape(1, _NUM_SPH * _NUM_RAD).astype(np.float32)
_NRMFLAT = _NRM.reshape(1, _NUM_SPH * _NUM_RAD).astype(np.float32)
_LCOL = np.repeat(np.arange(_NUM_SPH), _NUM_RAD)
_LEQ = [( _LCOL == l ).reshape(1, -1) for l in range(_NUM_SPH)]
_SPHC = [np.sqrt((2 * l + 1) / (4.0 * np.pi)).astype(np.float32)
         for l in range(_NUM_SPH)]


@functools.cache
def _make_sc_gather():
    @functools.partial(
        pl.kernel,
        mesh=plsc.VectorSubcoreMesh(core_axis_name="c", subcore_axis_name="s"),
        out_type=jax.ShapeDtypeStruct((_SC_NW, _NCHUNK, _CHUNK), jnp.float32),
        scratch_types=[
            pltpu.VMEM((_NCHUNK, _CHUNK), jnp.int32),
            pltpu.VMEM((_NCHUNK, _CHUNK), jnp.float32),
            pltpu.SemaphoreType.DMA,
        ],
    )
    def _sc_gather(table_hbm, idx_hbm, out_hbm, idx_v, rows_v, sem):
        wid = lax.axis_index("s") * _SC_NC + lax.axis_index("c")
        pltpu.sync_copy(idx_hbm.at[wid], idx_v)

        k = 10

        def body(g, carry):
            js = [g * k + t for t in range(k)]
            copies = [
                pltpu.async_copy(table_hbm.at[idx_v.at[j]], rows_v.at[j], sem)
                for j in js
            ]
            for cp in copies:
                cp.wait()
            return carry

        lax.fori_loop(0, _NCHUNK // k, body, 0)
        pltpu.sync_copy(rows_v, out_hbm.at[wid])

    return _sc_gather


def _tc_body(dg_ref, ang_ref, z_ref, nrm_ref, o_ref):
    dsr = dg_ref[...]
    th = ang_ref[...]
    zcol = z_ref[...]
    nrmcol = nrm_ref[...]
    nsr = _NUM_SPH * _NUM_RAD
    lrow = lax.broadcasted_iota(jnp.int32, (nsr, 1), 0) // _NUM_RAD
    leq = [lrow == l for l in range(_NUM_SPH)]

    inv_cutoff = 1.0 / _CUTOFF
    ds = dsr * inv_cutoff

    p = _ENV_EXP + 1
    a = -(p + 1) * (p + 2) / 2.0
    b = p * (p + 2) * 1.0
    c = -p * (p + 1) / 2.0
    env = 1.0 / ds + a * ds ** (p - 1) + b * ds ** p + c * ds ** (p + 1)
    u = jnp.where(ds < 1.0, env, jnp.zeros_like(env))

    x = zcol * ds
    s = jnp.sin(x)
    cx = jnp.cos(x)
    j0 = s / x
    j1 = s / x ** 2 - cx / x
    res = jnp.where(leq[0], j0, j1)
    jm1, jc = j0, j1
    for ll in range(1, _NUM_SPH - 1):
        jm1, jc = jc, (2 * ll + 1) / x * jc - jm1
        res = jnp.where(leq[ll + 1], jc, res)

    norm_const = inv_cutoff ** 1.5
    rbf_env = u * (res * nrmcol * norm_const)

    ct = jnp.cos(th)
    P = [jnp.ones_like(ct), ct]
    for l in range(1, _NUM_SPH - 1):
        P.append(((2 * l + 1) * ct * P[l] - l * P[l - 1]) / (l + 1))
    ys = [_SPHC[l] * P[l] for l in range(_NUM_SPH)]
    ysel = jnp.broadcast_to(ys[_NUM_SPH - 1], (nsr, ys[0].shape[1]))
    for l in range(_NUM_SPH - 2, -1, -1):
        ysel = jnp.where(leq[l], ys[l], ysel)

    o_ref[...] = (rbf_env * ysel).T


_BT = 1024


def _tc_compute(dg2, ang2):
    n = dg2.shape[1]
    nsr = _NUM_SPH * _NUM_RAD
    grid = n // _BT
    return pl.pallas_call(
        _tc_body,
        grid=(grid,),
        in_specs=[
            pl.BlockSpec((1, _BT), lambda i: (0, i)),
            pl.BlockSpec((1, _BT), lambda i: (0, i)),
            pl.BlockSpec((nsr, 1), lambda i: (0, 0)),
            pl.BlockSpec((nsr, 1), lambda i: (0, 0)),
        ],
        out_specs=pl.BlockSpec((_BT, nsr), lambda i: (i, 0)),
        out_shape=jax.ShapeDtypeStruct((n, nsr), jnp.float32),
    )(dg2, ang2, jnp.asarray(_ZFLAT.reshape(nsr, 1)),
      jnp.asarray(_NRMFLAT.reshape(nsr, 1)))


def kernel(D_ca, Angle_cab, id3_reduce_ca, Kidx):
    idx = id3_reduce_ca.astype(jnp.int32).reshape(_SC_NW, _NCHUNK, _CHUNK)
    d_g = _make_sc_gather()(D_ca, idx)
    return _tc_compute(d_g.reshape(1, _N_TRIP), Angle_cab.reshape(1, _N_TRIP))
